# Initial kernel scaffold; baseline (speedup 1.0000x reference)
import jax
import jax.numpy as jnp
from jax.experimental import pallas as pl


def kernel(x, edge_index, W_l1, b_l1, W_r1, W_l2, b_l2, W_r2):
    raise NotImplementedError("write your pallas kernel here")


# speedup vs baseline: 5.7734x; 5.7734x over previous
import functools

import jax
import jax.numpy as jnp
from jax import lax
from jax.experimental import pallas as pl
from jax.experimental.pallas import tpu as pltpu
from jax.experimental.pallas import tpu_sc as plsc

N_NODES = 10000
NPAD = 10240
N_EDGES = 320000
D = 128
NC, NS = 2, 16
NW = NC * NS
CHUNK = 128
NCHUNKS = N_EDGES // CHUNK
ITERS = (NCHUNKS + NW - 1) // NW
ROWS_PER_TILE = NPAD // NS
NSTG = ROWS_PER_TILE // CHUNK
DEG_W = 16

_MESH = plsc.VectorSubcoreMesh(
    core_axis_name="c", subcore_axis_name="s", num_cores=NC, num_subcores=NS
)


def _seg_body(feats, src, dst, zfeat, out, idx_s, idx_d, rows, agg_sh, sem):
    c = lax.axis_index("c")
    s = lax.axis_index("s")
    w = s * NC + c

    r0 = s * ROWS_PER_TILE
    pltpu.sync_copy(zfeat, rows)
    for j in range(NSTG):
        pltpu.sync_copy(rows, agg_sh.at[pl.ds(r0 + j * CHUNK, CHUNK)])
    plsc.subcore_barrier()

    def body(i, carry):
        cid = w + i * NW

        @pl.when(cid < NCHUNKS)
        def _():
            base = cid * CHUNK
            pltpu.sync_copy(src.at[pl.ds(base, CHUNK)], idx_s)
            pltpu.sync_copy(dst.at[pl.ds(base, CHUNK)], idx_d)
            pltpu.async_copy(feats.at[idx_s], rows, sem).wait()
            pltpu.sync_copy(rows, agg_sh.at[idx_d], add=True)

        return carry

    lax.fori_loop(0, ITERS, body, 0)
    plsc.subcore_barrier()

    for j in range(NSTG):
        sl = pl.ds(r0 + j * CHUNK, CHUNK)
        pltpu.sync_copy(agg_sh.at[sl], rows)
        pltpu.sync_copy(rows, out.at[c, sl])


def _deg_body(dst, zfeat, ones_h, degout, idx_d, ones_v, stg, deg_sh, sem):
    c = lax.axis_index("c")
    s = lax.axis_index("s")
    w = s * NC + c
    r0 = s * ROWS_PER_TILE
    pltpu.sync_copy(zfeat, stg)
    for j in range(NSTG):
        pltpu.sync_copy(stg, deg_sh.at[pl.ds(r0 + j * CHUNK, CHUNK)])
    pltpu.sync_copy(ones_h, ones_v)
    plsc.subcore_barrier()

    def body(i, carry):
        cid = w + i * NW

        @pl.when(cid < NCHUNKS)
        def _():
            base = cid * CHUNK
            pltpu.sync_copy(dst.at[pl.ds(base, CHUNK)], idx_d)
            pltpu.sync_copy(ones_v, deg_sh.at[idx_d], add=True)

        return carry

    lax.fori_loop(0, ITERS, body, 0)
    plsc.subcore_barrier()
    for j in range(NSTG):
        sl = pl.ds(r0 + j * CHUNK, CHUNK)
        pltpu.sync_copy(deg_sh.at[sl], stg)
        pltpu.sync_copy(stg, degout.at[c, sl])


_sc_segsum = functools.partial(
    pl.kernel,
    _seg_body,
    out_type=jax.ShapeDtypeStruct((NC, NPAD, D), jnp.float32),
    mesh=_MESH,
    scratch_types=[
        pltpu.VMEM((CHUNK,), jnp.int32),
        pltpu.VMEM((CHUNK,), jnp.int32),
        pltpu.VMEM((CHUNK, D), jnp.float32),
        pltpu.VMEM_SHARED((NPAD, D), jnp.float32),
        pltpu.SemaphoreType.DMA,
    ],
)()

_sc_deg = functools.partial(
    pl.kernel,
    _deg_body,
    out_type=jax.ShapeDtypeStruct((NC, NPAD, D), jnp.float32),
    mesh=_MESH,
    scratch_types=[
        pltpu.VMEM((CHUNK,), jnp.int32),
        pltpu.VMEM((CHUNK, D), jnp.float32),
        pltpu.VMEM((CHUNK, D), jnp.float32),
        pltpu.VMEM_SHARED((NPAD, D), jnp.float32),
        pltpu.SemaphoreType.DMA,
    ],
)()


def _dense_body(relu, aggp, degp, x, wl, b, wr, o):
    p = aggp[0] + aggp[1]
    deg = (jnp.sum(degp[0], axis=1) + jnp.sum(degp[1], axis=1)) * (1.0 / D)
    deg = jnp.maximum(deg, 1.0)
    mean = p / deg[:, None]
    acc = lax.dot_general(mean, wl[...], (((1,), (1,)), ((), ())),
                          preferred_element_type=jnp.float32)
    acc += lax.dot_general(x[...], wr[...], (((1,), (1,)), ((), ())),
                           preferred_element_type=jnp.float32)
    acc += b[...]
    o[...] = jnp.maximum(acc, 0.0) if relu else acc


def _dense(aggp, degp, x, wl, b, wr, relu):
    grid_n = 10
    r = NPAD // grid_n
    return pl.pallas_call(
        functools.partial(_dense_body, relu),
        out_shape=jax.ShapeDtypeStruct((NPAD, D), jnp.float32),
        grid=(grid_n,),
        in_specs=[
            pl.BlockSpec((NC, r, D), lambda i: (0, i, 0)),
            pl.BlockSpec((NC, r, D), lambda i: (0, i, 0)),
            pl.BlockSpec((r, D), lambda i: (i, 0)),
            pl.BlockSpec((D, D), lambda i: (0, 0)),
            pl.BlockSpec((1, D), lambda i: (0, 0)),
            pl.BlockSpec((D, D), lambda i: (0, 0)),
        ],
        out_specs=pl.BlockSpec((r, D), lambda i: (i, 0)),
    )(aggp, degp, x, wl, b, wr)


def kernel(x, edge_index, W_l1, b_l1, W_r1, W_l2, b_l2, W_r2):
    src = edge_index[0].astype(jnp.int32)
    dst = edge_index[1].astype(jnp.int32)
    xp = jnp.pad(x, ((0, NPAD - N_NODES), (0, 0)))
    zfeat = jnp.zeros((CHUNK, D), jnp.float32)
    ones_h = jnp.ones((CHUNK, D), jnp.float32)

    degp = _sc_deg(dst, zfeat, ones_h)
    aggp1 = _sc_segsum(xp, src, dst, zfeat)
    h = _dense(aggp1, degp, xp, W_l1, b_l1.reshape(1, -1), W_r1, relu=True)
    aggp2 = _sc_segsum(h, src, dst, zfeat)
    out = _dense(aggp2, degp, h, W_l2, b_l2.reshape(1, -1), W_r2, relu=False)
    return out[:N_NODES]
